# Initial kernel scaffold; baseline (speedup 1.0000x reference)
import functools

import jax
import jax.numpy as jnp
from jax import lax
from jax.experimental import pallas as pl
from jax.experimental.pallas import tpu as pltpu
from jax.experimental.pallas import tpu_sc as plsc

_NUM_CORES = 2
_NUM_SUBCORES = 16
_NW = _NUM_CORES * _NUM_SUBCORES
_CH = 80


def _make_sc_aggregate(n, d, nch, with_count):
    mesh = plsc.VectorSubcoreMesh(core_axis_name="c", subcore_axis_name="s")
    outs = [jax.ShapeDtypeStruct((_NUM_CORES, n, d), jnp.float32)]
    if with_count:
        outs.append(jax.ShapeDtypeStruct((_NUM_CORES, n, 16), jnp.float32))
    scratch = [
        pltpu.VMEM((nch, _CH), jnp.int32),
        pltpu.VMEM((nch, _CH), jnp.int32),
        pltpu.VMEM((_CH, d), jnp.float32),
    ]
    if with_count:
        scratch.append(pltpu.VMEM((_CH, 16), jnp.float32))
    scratch.append(pltpu.VMEM_SHARED((n, d), jnp.float32))
    if with_count:
        scratch.append(pltpu.VMEM_SHARED((n, 16), jnp.float32))
    scratch.append(pltpu.SemaphoreType.DMA)

    rows_per_sub = n // _NUM_SUBCORES

    def body(h_hbm, ei_hbm, z_d_hbm, *rest):
        if with_count:
            (z16_hbm, ones_hbm, agg_out, cnt_out,
             src_v, dst_v, rows_v, ones_v, sh_agg, sh_cnt, sem) = rest
        else:
            agg_out, src_v, dst_v, rows_v, sh_agg, sem = rest
        c = lax.axis_index("c")
        s = lax.axis_index("s")
        w = c * _NUM_SUBCORES + s
        sl = pl.ds(s * rows_per_sub, rows_per_sub)
        pltpu.sync_copy(z_d_hbm.at[sl], sh_agg.at[sl])
        if with_count:
            pltpu.sync_copy(z16_hbm.at[sl], sh_cnt.at[sl])
            pltpu.sync_copy(ones_hbm, ones_v)
        pltpu.sync_copy(ei_hbm.at[0, w], src_v)
        pltpu.sync_copy(ei_hbm.at[1, w], dst_v)
        plsc.subcore_barrier()

        @pl.loop(0, nch)
        def _(j):
            pltpu.async_copy(h_hbm.at[src_v.at[j]], rows_v, sem).wait()
            pltpu.sync_copy(rows_v, sh_agg.at[dst_v.at[j]], add=True)
            if with_count:
                pltpu.sync_copy(ones_v, sh_cnt.at[dst_v.at[j]], add=True)

        plsc.subcore_barrier()
        pltpu.sync_copy(sh_agg.at[sl], agg_out.at[c, sl])
        if with_count:
            pltpu.sync_copy(sh_cnt.at[sl], cnt_out.at[c, sl])

    return pl.kernel(body, mesh=mesh, out_type=tuple(outs),
                     scratch_types=scratch)


def _tc_layer_body(p_ref, c_ref, h_ref, wl_ref, bl_ref, wr_ref, o_ref):
    agg = p_ref[0] + p_ref[1]
    cnt = c_ref[0, :, 0:1] + c_ref[1, :, 0:1]
    mean = agg / jnp.maximum(cnt, 1.0)
    dn = (((1,), (1,)), ((), ()))
    out = lax.dot_general(mean, wl_ref[...], dn,
                          preferred_element_type=jnp.float32,
                          precision=lax.Precision.HIGHEST)
    out += lax.dot_general(h_ref[...], wr_ref[...], dn,
                           preferred_element_type=jnp.float32,
                           precision=lax.Precision.HIGHEST)
    out += bl_ref[...]
    o_ref[...] = jnp.maximum(out, 0.0)


def _tc_layer(parts, cnts, h, wl, bl, wr, blk):
    n, d = h.shape
    grid = (n // blk,)
    return pl.pallas_call(
        _tc_layer_body,
        grid=grid,
        in_specs=[
            pl.BlockSpec((_NUM_CORES, blk, d), lambda i: (0, i, 0)),
            pl.BlockSpec((_NUM_CORES, blk, 16), lambda i: (0, i, 0)),
            pl.BlockSpec((blk, d), lambda i: (i, 0)),
            pl.BlockSpec(wl.shape, lambda i: (0, 0)),
            pl.BlockSpec((1, wl.shape[0]), lambda i: (0, 0)),
            pl.BlockSpec(wr.shape, lambda i: (0, 0)),
        ],
        out_specs=pl.BlockSpec((blk, wl.shape[0]), lambda i: (i, 0)),
        out_shape=jax.ShapeDtypeStruct((n, wl.shape[0]), jnp.float32),
    )(parts, cnts, h, wl, bl.reshape(1, -1), wr)


def _sc_single(out):
    return out[0] if isinstance(out, (tuple, list)) else out


def kernel(x, edge_index, Wl1, bl1, Wr1, Wl2, bl2, Wr2):
    n, d = x.shape
    e = edge_index.shape[1]
    assert e % (_NW * _CH) == 0 and n % _NUM_SUBCORES == 0
    nch = e // (_NW * _CH)
    ei = edge_index.reshape(2, _NW, nch, _CH)
    z_d = jnp.zeros((n, d), jnp.float32)
    z16 = jnp.zeros((n, 16), jnp.float32)
    ones = jnp.ones((_CH, 16), jnp.float32)

    agg1 = _make_sc_aggregate(n, d, nch, True)
    agg2 = _make_sc_aggregate(n, d, nch, False)

    parts1, cnts = agg1(x, ei, z_d, z16, ones)
    h1 = _tc_layer(parts1, cnts, x, Wl1, bl1, Wr1, 2000)
    parts2 = _sc_single(agg2(h1, ei, z_d))
    h2 = _tc_layer(parts2, cnts, h1, Wl2, bl2, Wr2, 2000)
    return h2


# speedup vs baseline: 6.9550x; 6.9550x over previous
import jax
import jax.numpy as jnp
from jax import lax
from jax.experimental import pallas as pl
from jax.experimental.pallas import tpu as pltpu
from jax.experimental.pallas import tpu_sc as plsc

_NUM_CORES = 2
_NUM_SUBCORES = 16
_NW = _NUM_CORES * _NUM_SUBCORES
_CH = 80


def _make_sc_aggregate(n, d, nch):
    mesh = plsc.VectorSubcoreMesh(core_axis_name="c", subcore_axis_name="s")
    out = jax.ShapeDtypeStruct((_NUM_CORES, n, d), jnp.float32)
    scratch = [
        pltpu.VMEM((nch, _CH), jnp.int32),
        pltpu.VMEM((nch, _CH), jnp.int32),
        pltpu.VMEM((_CH, d), jnp.float32),
        pltpu.VMEM_SHARED((n, d), jnp.float32),
        pltpu.SemaphoreType.DMA,
    ]
    rows_per_sub = n // _NUM_SUBCORES

    def body(h_hbm, ei_hbm, z_hbm, agg_out, src_v, dst_v, rows_v, sh_agg, sem):
        c = lax.axis_index("c")
        s = lax.axis_index("s")
        w = c * _NUM_SUBCORES + s
        sl = pl.ds(s * rows_per_sub, rows_per_sub)
        pltpu.sync_copy(z_hbm.at[sl], sh_agg.at[sl])
        pltpu.sync_copy(ei_hbm.at[0, w], src_v)
        pltpu.sync_copy(ei_hbm.at[1, w], dst_v)
        plsc.subcore_barrier()

        @pl.loop(0, nch)
        def _(j):
            pltpu.async_copy(h_hbm.at[src_v.at[j]], rows_v, sem).wait()
            pltpu.sync_copy(rows_v, sh_agg.at[dst_v.at[j]], add=True)

        plsc.subcore_barrier()
        pltpu.sync_copy(sh_agg.at[sl], agg_out.at[c, sl])

    return pl.kernel(body, mesh=mesh, out_type=out, scratch_types=scratch)


def _make_sc_count(n, nch):
    mesh = plsc.VectorSubcoreMesh(core_axis_name="c", subcore_axis_name="s")
    out = jax.ShapeDtypeStruct((_NUM_CORES, n, 128), jnp.float32)
    scratch = [
        pltpu.VMEM((nch, _CH), jnp.int32),
        pltpu.VMEM((_CH, 128), jnp.float32),
        pltpu.VMEM_SHARED((n, 128), jnp.float32),
        pltpu.SemaphoreType.DMA,
    ]
    rows_per_sub = n // _NUM_SUBCORES

    def body(ei_hbm, z_hbm, ones_hbm, cnt_out, dst_v, ones_v, sh_cnt, sem):
        c = lax.axis_index("c")
        s = lax.axis_index("s")
        w = c * _NUM_SUBCORES + s
        sl = pl.ds(s * rows_per_sub, rows_per_sub)
        pltpu.sync_copy(z_hbm.at[sl], sh_cnt.at[sl])
        pltpu.sync_copy(ones_hbm, ones_v)
        pltpu.sync_copy(ei_hbm.at[1, w], dst_v)
        plsc.subcore_barrier()

        @pl.loop(0, nch)
        def _(j):
            pltpu.sync_copy(ones_v, sh_cnt.at[dst_v.at[j]], add=True)

        plsc.subcore_barrier()
        pltpu.sync_copy(sh_cnt.at[sl], cnt_out.at[c, sl])

    return pl.kernel(body, mesh=mesh, out_type=out, scratch_types=scratch)


def _tc_layer_body(p_ref, c_ref, h_ref, wl_ref, bl_ref, wr_ref, o_ref):
    agg = p_ref[0] + p_ref[1]
    cnt = c_ref[0, :, 0:1] + c_ref[1, :, 0:1]
    mean = agg / jnp.maximum(cnt, 1.0)
    dn = (((1,), (1,)), ((), ()))
    out = lax.dot_general(mean, wl_ref[...], dn,
                          preferred_element_type=jnp.float32,
                          precision=lax.Precision.HIGHEST)
    out += lax.dot_general(h_ref[...], wr_ref[...], dn,
                           preferred_element_type=jnp.float32,
                           precision=lax.Precision.HIGHEST)
    out += bl_ref[...]
    o_ref[...] = jnp.maximum(out, 0.0)


def _tc_layer(parts, cnts, h, wl, bl, wr, blk):
    n, d = h.shape
    grid = (n // blk,)
    return pl.pallas_call(
        _tc_layer_body,
        grid=grid,
        in_specs=[
            pl.BlockSpec((_NUM_CORES, blk, d), lambda i: (0, i, 0)),
            pl.BlockSpec((_NUM_CORES, blk, 128), lambda i: (0, i, 0)),
            pl.BlockSpec((blk, d), lambda i: (i, 0)),
            pl.BlockSpec(wl.shape, lambda i: (0, 0)),
            pl.BlockSpec((1, wl.shape[0]), lambda i: (0, 0)),
            pl.BlockSpec(wr.shape, lambda i: (0, 0)),
        ],
        out_specs=pl.BlockSpec((blk, wl.shape[0]), lambda i: (i, 0)),
        out_shape=jax.ShapeDtypeStruct((n, wl.shape[0]), jnp.float32),
    )(parts, cnts, h, wl, bl.reshape(1, -1), wr)


def kernel(x, edge_index, Wl1, bl1, Wr1, Wl2, bl2, Wr2):
    n, d = x.shape
    e = edge_index.shape[1]
    assert e % (_NW * _CH) == 0
    nch = e // (_NW * _CH)
    np_ = ((n + _NUM_SUBCORES * 8 - 1) // (_NUM_SUBCORES * 8)) * (_NUM_SUBCORES * 8)
    x_pad = jnp.concatenate([x, jnp.zeros((np_ - n, d), jnp.float32)]) if np_ != n else x
    ei = edge_index.reshape(2, _NW, nch, _CH)
    z_d = jnp.zeros((np_, d), jnp.float32)
    ones = jnp.ones((_CH, 128), jnp.float32)

    sc_agg = _make_sc_aggregate(np_, d, nch)
    sc_cnt = _make_sc_count(np_, nch)

    blk = 1264
    cnts = sc_cnt(ei, z_d, ones)
    parts1 = sc_agg(x_pad, ei, z_d)
    h1 = _tc_layer(parts1, cnts, x_pad, Wl1, bl1, Wr1, blk)
    parts2 = sc_agg(h1, ei, z_d)
    h2 = _tc_layer(parts2, cnts, h1, Wl2, bl2, Wr2, blk)
    return h2[:n]


# speedup vs baseline: 12.7004x; 1.8261x over previous
import jax
import jax.numpy as jnp
from jax import lax
from jax.experimental import pallas as pl
from jax.experimental.pallas import tpu as pltpu
from jax.experimental.pallas import tpu_sc as plsc

_NUM_CORES = 2
_NUM_SUBCORES = 16
_NW = _NUM_CORES * _NUM_SUBCORES
_CH = 80


def _make_sc_aggregate(n, d, np_, epw):
    nch = epw // _CH
    assert nch % 2 == 1, "pipeline below needs an odd chunk count"
    mesh = plsc.VectorSubcoreMesh(core_axis_name="c", subcore_axis_name="s")
    out = jax.ShapeDtypeStruct((_NUM_CORES, np_, d), jnp.float32)
    scratch = [
        pltpu.VMEM((epw,), jnp.int32),
        pltpu.VMEM((epw,), jnp.int32),
        pltpu.VMEM((_CH, d), jnp.float32),
        pltpu.VMEM((_CH, d), jnp.float32),
        pltpu.VMEM_SHARED((np_, d), jnp.float32),
        pltpu.SemaphoreType.DMA,
        pltpu.SemaphoreType.DMA,
    ]
    rows_per_sub = np_ // _NUM_SUBCORES

    def body(h_hbm, ei_hbm, z_hbm, agg_out,
             src_v, dst_v, rows_a, rows_b, sh_agg, sem_a, sem_b):
        c = lax.axis_index("c")
        s = lax.axis_index("s")
        w = c * _NUM_SUBCORES + s
        sl = pl.ds(s * rows_per_sub, rows_per_sub)
        pltpu.sync_copy(z_hbm.at[sl], sh_agg.at[sl])
        pltpu.sync_copy(ei_hbm.at[0, w, 0], src_v)
        pltpu.sync_copy(ei_hbm.at[1, w, 0], dst_v)
        plsc.subcore_barrier()

        bufs = (rows_a, rows_b)
        sems = (sem_a, sem_b)

        def start(j, b):
            pltpu.async_copy(h_hbm.at[src_v.at[pl.ds(j * _CH, _CH)]],
                             bufs[b], sems[b])

        def finish(b):
            pltpu.make_async_copy(h_hbm.at[src_v.at[pl.ds(0, _CH)]],
                                  bufs[b], sems[b]).wait()

        def scat(j, b):
            pltpu.sync_copy(bufs[b],
                            sh_agg.at[dst_v.at[pl.ds(j * _CH, _CH)]],
                            add=True)

        start(0, 0)

        @pl.loop(0, (nch - 1) // 2)
        def _(t):
            start(2 * t + 1, 1)
            finish(0)
            scat(2 * t, 0)
            start(2 * t + 2, 0)
            finish(1)
            scat(2 * t + 1, 1)

        finish(0)
        scat(nch - 1, 0)
        plsc.subcore_barrier()
        pltpu.sync_copy(sh_agg.at[sl], agg_out.at[c, sl])

    return pl.kernel(body, mesh=mesh, out_type=out, scratch_types=scratch)


def _make_sc_count(np_, epw):
    mesh = plsc.VectorSubcoreMesh(core_axis_name="c", subcore_axis_name="s")
    out = jax.ShapeDtypeStruct((_NW, 1, np_), jnp.float32)
    scratch = [
        pltpu.VMEM((epw,), jnp.int32),
        pltpu.VMEM((np_,), jnp.float32),
        pltpu.SemaphoreType.DMA,
    ]
    cp = pltpu.CompilerParams(needs_layout_passes=False)

    def body(ei_hbm, cnt_out, dst_v, hist_v, sem):
        c = lax.axis_index("c")
        s = lax.axis_index("s")
        w = c * _NUM_SUBCORES + s
        pltpu.sync_copy(ei_hbm.at[1, w, 0], dst_v)

        @pl.loop(0, np_ // 16)
        def _(i):
            hist_v[pl.ds(i * 16, 16)] = jnp.zeros((16,), jnp.float32)

        ones = jnp.full((16,), 1.0, jnp.float32)

        @pl.loop(0, epw // 16)
        def _(k):
            idx = dst_v[pl.ds(k * 16, 16)]
            plsc.addupdate_scatter(hist_v, [idx], ones)

        pltpu.sync_copy(hist_v, cnt_out.at[w, 0])

    return pl.kernel(body, mesh=mesh, out_type=out, scratch_types=scratch,
                     compiler_params=cp)


def _tc_layer_body(p_ref, c_ref, h_ref, wl_ref, bl_ref, wr_ref, o_ref):
    agg = p_ref[0] + p_ref[1]
    cnt = jnp.maximum(jnp.sum(c_ref[...], axis=1, keepdims=True), 1.0)
    mean = agg * (1.0 / cnt)
    dn = (((1,), (1,)), ((), ()))
    out = lax.dot_general(mean, wl_ref[...], dn,
                          preferred_element_type=jnp.float32,
                          precision=lax.Precision.HIGHEST)
    out += lax.dot_general(h_ref[...], wr_ref[...], dn,
                           preferred_element_type=jnp.float32,
                           precision=lax.Precision.HIGHEST)
    out += bl_ref[...]
    o_ref[...] = jnp.maximum(out, 0.0)


def _tc_layer(parts, cnts, h, wl, bl, wr, blk):
    n, d = h.shape
    grid = (n // blk,)
    return pl.pallas_call(
        _tc_layer_body,
        grid=grid,
        in_specs=[
            pl.BlockSpec((_NUM_CORES, blk, d), lambda i: (0, i, 0)),
            pl.BlockSpec((blk, _NW), lambda i: (i, 0)),
            pl.BlockSpec((blk, d), lambda i: (i, 0)),
            pl.BlockSpec(wl.shape, lambda i: (0, 0)),
            pl.BlockSpec((1, wl.shape[0]), lambda i: (0, 0)),
            pl.BlockSpec(wr.shape, lambda i: (0, 0)),
        ],
        out_specs=pl.BlockSpec((blk, wl.shape[0]), lambda i: (i, 0)),
        out_shape=jax.ShapeDtypeStruct((n, wl.shape[0]), jnp.float32),
    )(parts, cnts, h, wl, bl.reshape(1, -1), wr)


def kernel(x, edge_index, Wl1, bl1, Wr1, Wl2, bl2, Wr2):
    n, d = x.shape
    e = edge_index.shape[1]
    epw = e // _NW
    assert e % _NW == 0 and epw % _CH == 0 and epw % 16 == 0
    np_ = ((n + _NUM_SUBCORES * 8 - 1) // (_NUM_SUBCORES * 8)) * (_NUM_SUBCORES * 8)
    ei = edge_index.reshape(2, _NW, 1, epw)
    z_d = jnp.zeros((np_, d), jnp.float32)

    sc_agg = _make_sc_aggregate(n, d, np_, epw)
    sc_cnt = _make_sc_count(np_, epw)

    blk = 2000
    cnts = sc_cnt(ei).reshape(_NW, np_).T
    parts1 = sc_agg(x, ei, z_d)
    h1 = _tc_layer(parts1, cnts, x, Wl1, bl1, Wr1, blk)
    parts2 = sc_agg(h1, ei, z_d)
    h2 = _tc_layer(parts2, cnts, h1, Wl2, bl2, Wr2, blk)
    return h2


# speedup vs baseline: 13.6819x; 1.0773x over previous
import jax
import jax.numpy as jnp
from jax import lax
from jax.experimental import pallas as pl
from jax.experimental.pallas import tpu as pltpu
from jax.experimental.pallas import tpu_sc as plsc

_NUM_CORES = 2
_NUM_SUBCORES = 16
_NW = _NUM_CORES * _NUM_SUBCORES
_CH = 40
_K = 5
_G = 3


def _make_sc_aggregate(n, d, np_, epw):
    nch = epw // _CH
    assert nch % _K == 0, "ring schedule below needs nch % _K == 0"
    mesh = plsc.VectorSubcoreMesh(core_axis_name="c", subcore_axis_name="s")
    out = jax.ShapeDtypeStruct((_NUM_CORES, np_, d), jnp.float32)
    scratch = [
        pltpu.VMEM((epw,), jnp.int32),
        pltpu.VMEM((epw,), jnp.int32),
    ]
    scratch += [pltpu.VMEM((_CH, d), jnp.float32) for _ in range(_K)]
    scratch.append(pltpu.VMEM_SHARED((np_, d), jnp.float32))
    scratch += [pltpu.SemaphoreType.DMA for _ in range(_K)]
    rows_per_sub = np_ // _NUM_SUBCORES

    def body(h_hbm, ei_hbm, z_hbm, agg_out, src_v, dst_v, *rest):
        bufs = rest[:_K]
        sh_agg = rest[_K]
        sems = rest[_K + 1:]
        c = lax.axis_index("c")
        s = lax.axis_index("s")
        w = c * _NUM_SUBCORES + s
        sl = pl.ds(s * rows_per_sub, rows_per_sub)
        pltpu.sync_copy(z_hbm.at[sl], sh_agg.at[sl])
        pltpu.sync_copy(ei_hbm.at[0, w, 0], src_v)
        pltpu.sync_copy(ei_hbm.at[1, w, 0], dst_v)
        plsc.subcore_barrier()

        def gather(j, b):
            pltpu.async_copy(h_hbm.at[src_v.at[pl.ds(j * _CH, _CH)]],
                             bufs[b], sems[b])

        def scatter(j, b):
            pltpu.async_copy(bufs[b],
                             sh_agg.at[dst_v.at[pl.ds(j * _CH, _CH)]],
                             sems[b], add=True)

        def finish(b):
            pltpu.make_async_copy(h_hbm.at[src_v.at[pl.ds(0, _CH)]],
                                  bufs[b], sems[b]).wait()

        for g in range(_G):
            gather(g, g)

        @pl.loop(0, nch // _K)
        def _(t):
            for b in range(_K):
                j = t * _K + b
                finish(b)
                scatter(j, b)
                bb = (b + _G) % _K
                nxt = j + _G

                @pl.when(nxt < nch)
                def _():
                    @pl.when(j >= _K - _G)
                    def _():
                        finish(bb)
                    gather(nxt, bb)

        for b in range(_K):
            finish(b)
        plsc.subcore_barrier()
        pltpu.sync_copy(sh_agg.at[sl], agg_out.at[c, sl])

    return pl.kernel(body, mesh=mesh, out_type=out, scratch_types=scratch)


def _make_sc_count(np_, epw):
    mesh = plsc.VectorSubcoreMesh(core_axis_name="c", subcore_axis_name="s")
    out = jax.ShapeDtypeStruct((_NW, 1, np_), jnp.float32)
    scratch = [
        pltpu.VMEM((epw,), jnp.int32),
        pltpu.VMEM((np_,), jnp.float32),
        pltpu.SemaphoreType.DMA,
    ]
    cp = pltpu.CompilerParams(needs_layout_passes=False)

    def body(ei_hbm, cnt_out, dst_v, hist_v, sem):
        c = lax.axis_index("c")
        s = lax.axis_index("s")
        w = c * _NUM_SUBCORES + s
        pltpu.sync_copy(ei_hbm.at[1, w, 0], dst_v)

        @pl.loop(0, np_ // 16)
        def _(i):
            hist_v[pl.ds(i * 16, 16)] = jnp.zeros((16,), jnp.float32)

        ones = jnp.full((16,), 1.0, jnp.float32)

        @pl.loop(0, epw // 16)
        def _(k):
            idx = dst_v[pl.ds(k * 16, 16)]
            plsc.addupdate_scatter(hist_v, [idx], ones)

        pltpu.sync_copy(hist_v, cnt_out.at[w, 0])

    return pl.kernel(body, mesh=mesh, out_type=out, scratch_types=scratch,
                     compiler_params=cp)


def _tc_layer_body(p_ref, c_ref, h_ref, wl_ref, bl_ref, wr_ref, o_ref):
    agg = p_ref[0] + p_ref[1]
    cnt = jnp.maximum(jnp.sum(c_ref[...], axis=1, keepdims=True), 1.0)
    mean = agg * (1.0 / cnt)
    dn = (((1,), (1,)), ((), ()))
    out = lax.dot_general(mean, wl_ref[...], dn,
                          preferred_element_type=jnp.float32,
                          precision=lax.Precision.HIGHEST)
    out += lax.dot_general(h_ref[...], wr_ref[...], dn,
                           preferred_element_type=jnp.float32,
                           precision=lax.Precision.HIGHEST)
    out += bl_ref[...]
    o_ref[...] = jnp.maximum(out, 0.0)


def _tc_layer(parts, cnts, h, wl, bl, wr, blk):
    n, d = h.shape
    grid = (n // blk,)
    return pl.pallas_call(
        _tc_layer_body,
        grid=grid,
        in_specs=[
            pl.BlockSpec((_NUM_CORES, blk, d), lambda i: (0, i, 0)),
            pl.BlockSpec((blk, _NW), lambda i: (i, 0)),
            pl.BlockSpec((blk, d), lambda i: (i, 0)),
            pl.BlockSpec(wl.shape, lambda i: (0, 0)),
            pl.BlockSpec((1, wl.shape[0]), lambda i: (0, 0)),
            pl.BlockSpec(wr.shape, lambda i: (0, 0)),
        ],
        out_specs=pl.BlockSpec((blk, wl.shape[0]), lambda i: (i, 0)),
        out_shape=jax.ShapeDtypeStruct((n, wl.shape[0]), jnp.float32),
    )(parts, cnts, h, wl, bl.reshape(1, -1), wr)


def kernel(x, edge_index, Wl1, bl1, Wr1, Wl2, bl2, Wr2):
    n, d = x.shape
    e = edge_index.shape[1]
    epw = e // _NW
    assert e % _NW == 0 and epw % _CH == 0 and epw % 16 == 0
    np_ = ((n + _NUM_SUBCORES * 8 - 1) // (_NUM_SUBCORES * 8)) * (_NUM_SUBCORES * 8)
    ei = edge_index.reshape(2, _NW, 1, epw)
    z_d = jnp.zeros((np_, d), jnp.float32)

    sc_agg = _make_sc_aggregate(n, d, np_, epw)
    sc_cnt = _make_sc_count(np_, epw)

    blk = 2000
    cnts = sc_cnt(ei).reshape(_NW, np_).T
    parts1 = sc_agg(x, ei, z_d)
    h1 = _tc_layer(parts1, cnts, x, Wl1, bl1, Wr1, blk)
    parts2 = sc_agg(h1, ei, z_d)
    h2 = _tc_layer(parts2, cnts, h1, Wl2, bl2, Wr2, blk)
    return h2


# speedup vs baseline: 14.7398x; 1.0773x over previous
import jax
import jax.numpy as jnp
from jax import lax
from jax.experimental import pallas as pl
from jax.experimental.pallas import tpu as pltpu
from jax.experimental.pallas import tpu_sc as plsc

_NUM_CORES = 2
_NUM_SUBCORES = 16
_NW = _NUM_CORES * _NUM_SUBCORES
_CH = 40
_K = 5
_G = 4


def _make_sc_aggregate(n, d, np_, epw):
    nch = epw // _CH
    assert nch % _K == 0, "ring schedule below needs nch % _K == 0"
    mesh = plsc.VectorSubcoreMesh(core_axis_name="c", subcore_axis_name="s")
    out = jax.ShapeDtypeStruct((_NUM_CORES, np_, d), jnp.float32)
    scratch = [
        pltpu.VMEM((epw,), jnp.int32),
        pltpu.VMEM((epw,), jnp.int32),
    ]
    scratch += [pltpu.VMEM((_CH, d), jnp.float32) for _ in range(_K)]
    scratch.append(pltpu.VMEM_SHARED((np_, d), jnp.float32))
    scratch += [pltpu.SemaphoreType.DMA for _ in range(_K)]
    rows_per_sub = np_ // _NUM_SUBCORES

    def body(h_hbm, ei_hbm, z_hbm, agg_out, src_v, dst_v, *rest):
        bufs = rest[:_K]
        sh_agg = rest[_K]
        sems = rest[_K + 1:]
        c = lax.axis_index("c")
        s = lax.axis_index("s")
        w = c * _NUM_SUBCORES + s
        sl = pl.ds(s * rows_per_sub, rows_per_sub)
        pltpu.sync_copy(z_hbm.at[sl], sh_agg.at[sl])
        pltpu.sync_copy(ei_hbm.at[0, w, 0], src_v)
        pltpu.sync_copy(ei_hbm.at[1, w, 0], dst_v)
        plsc.subcore_barrier()

        def gather(j, b):
            pltpu.async_copy(h_hbm.at[src_v.at[pl.ds(j * _CH, _CH)]],
                             bufs[b], sems[b])

        def scatter(j, b):
            pltpu.async_copy(bufs[b],
                             sh_agg.at[dst_v.at[pl.ds(j * _CH, _CH)]],
                             sems[b], add=True)

        def finish(b):
            pltpu.make_async_copy(h_hbm.at[src_v.at[pl.ds(0, _CH)]],
                                  bufs[b], sems[b]).wait()

        for g in range(_G):
            gather(g, g)

        @pl.loop(0, nch // _K)
        def _(t):
            for b in range(_K):
                j = t * _K + b
                finish(b)
                scatter(j, b)
                bb = (b + _G) % _K
                nxt = j + _G

                @pl.when(nxt < nch)
                def _():
                    @pl.when(j >= _K - _G)
                    def _():
                        finish(bb)
                    gather(nxt, bb)

        for b in range(_K):
            finish(b)
        plsc.subcore_barrier()
        pltpu.sync_copy(sh_agg.at[sl], agg_out.at[c, sl])

    return pl.kernel(body, mesh=mesh, out_type=out, scratch_types=scratch)


def _make_sc_count(np_, epw):
    mesh = plsc.VectorSubcoreMesh(core_axis_name="c", subcore_axis_name="s")
    out = jax.ShapeDtypeStruct((_NW, 1, np_), jnp.float32)
    scratch = [
        pltpu.VMEM((epw,), jnp.int32),
        pltpu.VMEM((np_,), jnp.float32),
        pltpu.SemaphoreType.DMA,
    ]
    cp = pltpu.CompilerParams(needs_layout_passes=False)

    def body(ei_hbm, cnt_out, dst_v, hist_v, sem):
        c = lax.axis_index("c")
        s = lax.axis_index("s")
        w = c * _NUM_SUBCORES + s
        pltpu.sync_copy(ei_hbm.at[1, w, 0], dst_v)

        @pl.loop(0, np_ // 16)
        def _(i):
            hist_v[pl.ds(i * 16, 16)] = jnp.zeros((16,), jnp.float32)

        ones = jnp.full((16,), 1.0, jnp.float32)

        @pl.loop(0, epw // 16)
        def _(k):
            idx = dst_v[pl.ds(k * 16, 16)]
            plsc.addupdate_scatter(hist_v, [idx], ones)

        pltpu.sync_copy(hist_v, cnt_out.at[w, 0])

    return pl.kernel(body, mesh=mesh, out_type=out, scratch_types=scratch,
                     compiler_params=cp)


def _tc_layer_body(p_ref, c_ref, h_ref, wl_ref, bl_ref, wr_ref, o_ref):
    agg = p_ref[0] + p_ref[1]
    cnt = jnp.maximum(jnp.sum(c_ref[...], axis=1, keepdims=True), 1.0)
    mean = agg * (1.0 / cnt)
    dn = (((1,), (1,)), ((), ()))
    out = lax.dot_general(mean, wl_ref[...], dn,
                          preferred_element_type=jnp.float32,
                          precision=lax.Precision.HIGHEST)
    out += lax.dot_general(h_ref[...], wr_ref[...], dn,
                           preferred_element_type=jnp.float32,
                           precision=lax.Precision.HIGHEST)
    out += bl_ref[...]
    o_ref[...] = jnp.maximum(out, 0.0)


def _tc_layer(parts, cnts, h, wl, bl, wr, blk):
    n, d = h.shape
    grid = (n // blk,)
    return pl.pallas_call(
        _tc_layer_body,
        grid=grid,
        in_specs=[
            pl.BlockSpec((_NUM_CORES, blk, d), lambda i: (0, i, 0)),
            pl.BlockSpec((blk, _NW), lambda i: (i, 0)),
            pl.BlockSpec((blk, d), lambda i: (i, 0)),
            pl.BlockSpec(wl.shape, lambda i: (0, 0)),
            pl.BlockSpec((1, wl.shape[0]), lambda i: (0, 0)),
            pl.BlockSpec(wr.shape, lambda i: (0, 0)),
        ],
        out_specs=pl.BlockSpec((blk, wl.shape[0]), lambda i: (i, 0)),
        out_shape=jax.ShapeDtypeStruct((n, wl.shape[0]), jnp.float32),
    )(parts, cnts, h, wl, bl.reshape(1, -1), wr)


def kernel(x, edge_index, Wl1, bl1, Wr1, Wl2, bl2, Wr2):
    n, d = x.shape
    e = edge_index.shape[1]
    epw = e // _NW
    assert e % _NW == 0 and epw % _CH == 0 and epw % 16 == 0
    np_ = ((n + _NUM_SUBCORES * 8 - 1) // (_NUM_SUBCORES * 8)) * (_NUM_SUBCORES * 8)
    ei = edge_index.reshape(2, _NW, 1, epw)
    z_d = jnp.zeros((np_, d), jnp.float32)

    sc_agg = _make_sc_aggregate(n, d, np_, epw)
    sc_cnt = _make_sc_count(np_, epw)

    blk = 2000
    cnts = sc_cnt(ei).reshape(_NW, np_).T
    parts1 = sc_agg(x, ei, z_d)
    h1 = _tc_layer(parts1, cnts, x, Wl1, bl1, Wr1, blk)
    parts2 = sc_agg(h1, ei, z_d)
    h2 = _tc_layer(parts2, cnts, h1, Wl2, bl2, Wr2, blk)
    return h2


# speedup vs baseline: 15.0365x; 1.0201x over previous
import jax
import jax.numpy as jnp
from jax import lax
from jax.experimental import pallas as pl
from jax.experimental.pallas import tpu as pltpu
from jax.experimental.pallas import tpu_sc as plsc

_NUM_CORES = 2
_NUM_SUBCORES = 16
_NW = _NUM_CORES * _NUM_SUBCORES
_CH = 40
_K = 5
_G = 4


def _make_sc_aggregate(n, d, np_, epw):
    nch = epw // _CH
    assert nch % _K == 0, "ring schedule below needs nch % _K == 0"
    mesh = plsc.VectorSubcoreMesh(core_axis_name="c", subcore_axis_name="s")
    out = jax.ShapeDtypeStruct((_NUM_CORES, np_, d), jnp.float32)
    scratch = [
        pltpu.VMEM((epw,), jnp.int32),
        pltpu.VMEM((epw,), jnp.int32),
    ]
    scratch += [pltpu.VMEM((_CH, d), jnp.float32) for _ in range(_K)]
    scratch.append(pltpu.VMEM_SHARED((np_, d), jnp.float32))
    scratch += [pltpu.SemaphoreType.DMA for _ in range(_K)]
    rows_per_sub = np_ // _NUM_SUBCORES

    def body(h_hbm, ei_hbm, z_hbm, agg_out, src_v, dst_v, *rest):
        bufs = rest[:_K]
        sh_agg = rest[_K]
        sems = rest[_K + 1:]
        c = lax.axis_index("c")
        s = lax.axis_index("s")
        w = c * _NUM_SUBCORES + s
        sl = pl.ds(s * rows_per_sub, rows_per_sub)
        zero_cp = pltpu.async_copy(z_hbm.at[sl], sh_agg.at[sl], sems[0])
        src_cp = pltpu.async_copy(ei_hbm.at[0, w, 0], src_v, sems[1])
        dst_cp = pltpu.async_copy(ei_hbm.at[1, w, 0], dst_v, sems[2])
        zero_cp.wait()
        src_cp.wait()
        dst_cp.wait()
        plsc.subcore_barrier()

        def gather(j, b):
            pltpu.async_copy(h_hbm.at[src_v.at[pl.ds(j * _CH, _CH)]],
                             bufs[b], sems[b])

        def scatter(j, b):
            pltpu.async_copy(bufs[b],
                             sh_agg.at[dst_v.at[pl.ds(j * _CH, _CH)]],
                             sems[b], add=True)

        def finish(b):
            pltpu.make_async_copy(h_hbm.at[src_v.at[pl.ds(0, _CH)]],
                                  bufs[b], sems[b]).wait()

        for g in range(_G):
            gather(g, g)

        @pl.loop(0, nch // _K)
        def _(t):
            for b in range(_K):
                j = t * _K + b
                finish(b)
                scatter(j, b)
                bb = (b + _G) % _K
                nxt = j + _G

                @pl.when(nxt < nch)
                def _():
                    @pl.when(j >= _K - _G)
                    def _():
                        finish(bb)
                    gather(nxt, bb)

        for b in range(_K):
            finish(b)
        plsc.subcore_barrier()
        pltpu.sync_copy(sh_agg.at[sl], agg_out.at[c, sl])

    return pl.kernel(body, mesh=mesh, out_type=out, scratch_types=scratch)


def _make_sc_count(np_, epw):
    mesh = plsc.VectorSubcoreMesh(core_axis_name="c", subcore_axis_name="s")
    out = jax.ShapeDtypeStruct((_NW, 1, np_), jnp.float32)
    scratch = [
        pltpu.VMEM((epw,), jnp.int32),
        pltpu.VMEM((np_,), jnp.float32),
        pltpu.SemaphoreType.DMA,
    ]
    cp = pltpu.CompilerParams(needs_layout_passes=False)

    def body(ei_hbm, cnt_out, dst_v, hist_v, sem):
        c = lax.axis_index("c")
        s = lax.axis_index("s")
        w = c * _NUM_SUBCORES + s
        dst_cp = pltpu.async_copy(ei_hbm.at[1, w, 0], dst_v, sem)

        @pl.loop(0, np_ // 16)
        def _(i):
            hist_v[pl.ds(i * 16, 16)] = jnp.zeros((16,), jnp.float32)

        dst_cp.wait()

        ones = jnp.full((16,), 1.0, jnp.float32)

        @pl.loop(0, epw // 16)
        def _(k):
            idx = dst_v[pl.ds(k * 16, 16)]
            plsc.addupdate_scatter(hist_v, [idx], ones)

        pltpu.sync_copy(hist_v, cnt_out.at[w, 0])

    return pl.kernel(body, mesh=mesh, out_type=out, scratch_types=scratch,
                     compiler_params=cp)


def _tc_layer_body(p_ref, c_ref, h_ref, wl_ref, bl_ref, wr_ref, o_ref):
    agg = p_ref[0] + p_ref[1]
    cnt = jnp.maximum(jnp.sum(c_ref[...], axis=1, keepdims=True), 1.0)
    mean = agg * (1.0 / cnt)
    dn = (((1,), (1,)), ((), ()))
    out = lax.dot_general(mean, wl_ref[...], dn,
                          preferred_element_type=jnp.float32,
                          precision=lax.Precision.HIGHEST)
    out += lax.dot_general(h_ref[...], wr_ref[...], dn,
                           preferred_element_type=jnp.float32,
                           precision=lax.Precision.HIGHEST)
    out += bl_ref[...]
    o_ref[...] = jnp.maximum(out, 0.0)


def _tc_layer(parts, cnts, h, wl, bl, wr, blk):
    n, d = h.shape
    grid = (n // blk,)
    return pl.pallas_call(
        _tc_layer_body,
        grid=grid,
        in_specs=[
            pl.BlockSpec((_NUM_CORES, blk, d), lambda i: (0, i, 0)),
            pl.BlockSpec((blk, _NW), lambda i: (i, 0)),
            pl.BlockSpec((blk, d), lambda i: (i, 0)),
            pl.BlockSpec(wl.shape, lambda i: (0, 0)),
            pl.BlockSpec((1, wl.shape[0]), lambda i: (0, 0)),
            pl.BlockSpec(wr.shape, lambda i: (0, 0)),
        ],
        out_specs=pl.BlockSpec((blk, wl.shape[0]), lambda i: (i, 0)),
        out_shape=jax.ShapeDtypeStruct((n, wl.shape[0]), jnp.float32),
    )(parts, cnts, h, wl, bl.reshape(1, -1), wr)


def kernel(x, edge_index, Wl1, bl1, Wr1, Wl2, bl2, Wr2):
    n, d = x.shape
    e = edge_index.shape[1]
    epw = e // _NW
    assert e % _NW == 0 and epw % _CH == 0 and epw % 16 == 0
    np_ = ((n + _NUM_SUBCORES * 8 - 1) // (_NUM_SUBCORES * 8)) * (_NUM_SUBCORES * 8)
    ei = edge_index.reshape(2, _NW, 1, epw)
    z_d = jnp.zeros((np_, d), jnp.float32)

    sc_agg = _make_sc_aggregate(n, d, np_, epw)
    sc_cnt = _make_sc_count(np_, epw)

    blk = 2000
    cnts = sc_cnt(ei).reshape(_NW, np_).T
    parts1 = sc_agg(x, ei, z_d)
    h1 = _tc_layer(parts1, cnts, x, Wl1, bl1, Wr1, blk)
    parts2 = sc_agg(h1, ei, z_d)
    h2 = _tc_layer(parts2, cnts, h1, Wl2, bl2, Wr2, blk)
    return h2
